# baseline (speedup 1.0000x reference)
import functools

import jax
import jax.numpy as jnp
from jax import lax
from jax.experimental import pallas as pl
from jax.experimental.pallas import tpu as pltpu
from jax.experimental.pallas import tpu_sc as plsc

_NC = 2
_NS = 16
_NW = _NC * _NS
_L = 16

_B, _T, _D = 128, 1024, 8
_SC_ROWS = 32
_TC_ROWS = _B - _SC_ROWS
_RPW = _SC_ROWS // _NW
_NT = _T // _L
_TC_RB = 8

_MAGIC = 8388608.0


def _decode(x, sel_dtype):
    one, half, zero = 1.0, 0.5, 0.0
    D = len(x)

    s = [jnp.where(x[j] >= zero, _MAGIC, -_MAGIC) for j in range(D)]
    f0 = [(x[j] + s[j]) - s[j] for j in range(D)]
    d0 = [x[j] - f0[j] for j in range(D)]
    neg = [d0[j] < zero for j in range(D)]
    f1 = [f0[j] - jnp.where(neg[j], one, zero) for j in range(D)]
    d1 = [d0[j] + jnp.where(neg[j], half, -half) for j in range(D)]

    def coset(f, d):
        p = f[0] + f[1]
        for j in range(2, D):
            p = p + f[j]
        odd = (p.astype(jnp.int32) & 1) == 1
        ad = [jnp.abs(d[j]) for j in range(D)]
        best = ad[0]
        bidx = jnp.zeros(best.shape, sel_dtype)
        for j in range(1, D):
            gt = ad[j] > best
            best = jnp.maximum(best, ad[j])
            bidx = jnp.where(gt, jnp.asarray(j, sel_dtype), bidx)
        g_out = []
        for j in range(D):
            fix = (bidx == j) & odd
            sgn = jnp.where(d[j] >= zero, one, -one)
            g_out.append(f[j] + jnp.where(fix, sgn, zero))
        dist = d[0] * d[0]
        for j in range(1, D):
            dist = dist + d[j] * d[j]
        dist = dist + jnp.where(odd, one - (best + best), zero)
        return g_out, dist

    g0, dist0 = coset(f0, d0)
    g1, dist1 = coset(f1, d1)
    pick0 = dist0 <= dist1
    return [jnp.where(pick0, g0[j], g1[j] + half) for j in range(D)]


def _e8_sc_body(x_hbm, out_hbm, xv, ov):
    wid = lax.axis_index("s") * _NC + lax.axis_index("c")
    row0 = wid * _RPW
    pltpu.sync_copy(x_hbm.at[pl.ds(row0, _RPW)], xv)

    for b in range(_RPW):
        @plsc.parallel_loop(0, _NT, 1, unroll=2)
        def step(t, b=b):
            t0 = t * _L
            x = [xv[b, j, pl.ds(t0, _L)] for j in range(_D)]
            out = _decode(x, jnp.int32)
            for j in range(_D):
                ov[b, j, pl.ds(t0, _L)] = out[j]

    pltpu.sync_copy(ov, out_hbm.at[pl.ds(row0, _RPW)])


_e8_sc = functools.partial(
    pl.kernel,
    out_type=jax.ShapeDtypeStruct((_SC_ROWS, _D, _T), jnp.float32),
    mesh=plsc.VectorSubcoreMesh(core_axis_name="c", subcore_axis_name="s"),
    scratch_types=[
        pltpu.VMEM((_RPW, _D, _T), jnp.float32),
        pltpu.VMEM((_RPW, _D, _T), jnp.float32),
    ],
    compiler_params=pltpu.CompilerParams(needs_layout_passes=False),
)(_e8_sc_body)


def _e8_tc_body(x_ref, o_ref):
    xb = x_ref[...]
    x = [xb[:, j, :] for j in range(_D)]
    out = _decode(x, jnp.int32)
    o_ref[...] = jnp.stack(out, axis=1)


_e8_tc = pl.pallas_call(
    _e8_tc_body,
    out_shape=jax.ShapeDtypeStruct((_TC_ROWS, _D, _T), jnp.float32),
    grid=(_TC_ROWS // _TC_RB,),
    in_specs=[pl.BlockSpec((_TC_RB, _D, _T), lambda i: (i, 0, 0))],
    out_specs=pl.BlockSpec((_TC_RB, _D, _T), lambda i: (i, 0, 0)),
    compiler_params=pltpu.CompilerParams(
        dimension_semantics=("arbitrary",),
    ),
)


@jax.jit
def kernel(x):
    if x.shape[-1] != 8:
        raise ValueError(f"E8 expects [..., 8] input, got shape {x.shape}")
    xt = jnp.transpose(x, (0, 2, 1))
    y_sc = _e8_sc(xt[:_SC_ROWS])
    y_tc = _e8_tc(xt[_SC_ROWS:])
    y_t = jnp.concatenate([y_sc, y_tc], axis=0)
    return jnp.transpose(y_t, (0, 2, 1))


# speedup vs baseline: 1.7350x; 1.7350x over previous
import functools

import jax
import jax.numpy as jnp
from jax import lax
from jax.experimental import pallas as pl
from jax.experimental.pallas import tpu as pltpu
from jax.experimental.pallas import tpu_sc as plsc

_NC = 2
_NS = 16
_NW = _NC * _NS
_L = 16

_B, _T, _D = 128, 1024, 8
_SC_ROWS = 32
_TC_ROWS = _B - _SC_ROWS
_RPW = _SC_ROWS // _NW
_NT = _T // _L
_TC_RB = 8

_MAGIC = 8388608.0


def _decode(x, sel_dtype):
    one, half, zero = 1.0, 0.5, 0.0
    D = len(x)

    s = [jnp.where(x[j] >= zero, _MAGIC, -_MAGIC) for j in range(D)]
    f0 = [(x[j] + s[j]) - s[j] for j in range(D)]
    d0 = [x[j] - f0[j] for j in range(D)]
    neg = [d0[j] < zero for j in range(D)]
    f1 = [f0[j] - jnp.where(neg[j], one, zero) for j in range(D)]
    d1 = [d0[j] + jnp.where(neg[j], half, -half) for j in range(D)]

    def coset(f, d):
        p = f[0] + f[1]
        for j in range(2, D):
            p = p + f[j]
        odd = (p.astype(jnp.int32) & 1) == 1
        ad = [jnp.abs(d[j]) for j in range(D)]
        best = ad[0]
        bidx = jnp.zeros(best.shape, sel_dtype)
        for j in range(1, D):
            gt = ad[j] > best
            best = jnp.maximum(best, ad[j])
            bidx = jnp.where(gt, jnp.asarray(j, sel_dtype), bidx)
        g_out = []
        for j in range(D):
            fix = (bidx == j) & odd
            sgn = jnp.where(d[j] >= zero, one, -one)
            g_out.append(f[j] + jnp.where(fix, sgn, zero))
        dist = d[0] * d[0]
        for j in range(1, D):
            dist = dist + d[j] * d[j]
        dist = dist + jnp.where(odd, one - (best + best), zero)
        return g_out, dist

    g0, dist0 = coset(f0, d0)
    g1, dist1 = coset(f1, d1)
    pick0 = dist0 <= dist1
    return [jnp.where(pick0, g0[j], g1[j] + half) for j in range(D)]


def _e8_sc_body(x_hbm, out_hbm, xv, ov):
    wid = lax.axis_index("s") * _NC + lax.axis_index("c")
    row0 = wid * _RPW
    pltpu.sync_copy(x_hbm.at[pl.ds(row0, _RPW)], xv)

    for b in range(_RPW):
        @plsc.parallel_loop(0, _NT, 1, unroll=2)
        def step(t, b=b):
            t0 = t * _L
            x = [xv[b, j, pl.ds(t0, _L)] for j in range(_D)]
            out = _decode(x, jnp.int32)
            for j in range(_D):
                ov[b, j, pl.ds(t0, _L)] = out[j]

    pltpu.sync_copy(ov, out_hbm.at[pl.ds(row0, _RPW)])


_e8_sc = functools.partial(
    pl.kernel,
    out_type=jax.ShapeDtypeStruct((_SC_ROWS, _D, _T), jnp.float32),
    mesh=plsc.VectorSubcoreMesh(core_axis_name="c", subcore_axis_name="s"),
    scratch_types=[
        pltpu.VMEM((_RPW, _D, _T), jnp.float32),
        pltpu.VMEM((_RPW, _D, _T), jnp.float32),
    ],
    compiler_params=pltpu.CompilerParams(needs_layout_passes=False),
)(_e8_sc_body)


def _e8_tc_body(x_ref, o_ref):
    one, half, zero = 1.0, 0.5, 0.0
    x = x_ref[...]
    s = jnp.where(x >= zero, _MAGIC, -_MAGIC)
    f0 = (x + s) - s
    d0 = x - f0
    neg = d0 < zero
    f1 = f0 - jnp.where(neg, one, zero)
    d1 = d0 + jnp.where(neg, half, -half)
    jota = lax.broadcasted_iota(jnp.int32, x.shape, 1)

    def coset(f, d):
        p = jnp.sum(f, axis=1, keepdims=True)
        odd = (p.astype(jnp.int32) & 1) == 1
        ad = jnp.abs(d)
        m = jnp.max(ad, axis=1, keepdims=True)
        cand = jnp.where(ad == m, jota, _D)
        idx = jnp.min(cand, axis=1, keepdims=True)
        fix = (jota == idx) & odd
        sgn = jnp.where(d >= zero, one, -one)
        g = f + jnp.where(fix, sgn, zero)
        dist = jnp.sum(d * d, axis=1, keepdims=True)
        dist = dist + jnp.where(odd, one - (m + m), zero)
        return g, dist

    g0, dist0 = coset(f0, d0)
    g1, dist1 = coset(f1, d1)
    o_ref[...] = jnp.where(dist0 <= dist1, g0, g1 + half)


_TC_OFF = _SC_ROWS // _TC_RB

_e8_tc = pl.pallas_call(
    _e8_tc_body,
    out_shape=jax.ShapeDtypeStruct((_TC_ROWS, _D, _T), jnp.float32),
    grid=(_TC_ROWS // _TC_RB,),
    in_specs=[pl.BlockSpec((_TC_RB, _D, _T), lambda i: (i + _TC_OFF, 0, 0))],
    out_specs=pl.BlockSpec((_TC_RB, _D, _T), lambda i: (i, 0, 0)),
    compiler_params=pltpu.CompilerParams(
        dimension_semantics=("arbitrary",),
    ),
)


@jax.jit
def kernel(x):
    if x.shape[-1] != 8:
        raise ValueError(f"E8 expects [..., 8] input, got shape {x.shape}")
    xt = jnp.transpose(x, (0, 2, 1))
    y_sc = _e8_sc(xt)
    y_tc = _e8_tc(xt)
    y_t = jnp.concatenate([y_sc, y_tc], axis=0)
    return jnp.transpose(y_t, (0, 2, 1))


# speedup vs baseline: 1.7959x; 1.0351x over previous
import functools

import jax
import jax.numpy as jnp
from jax import lax
from jax.experimental import pallas as pl
from jax.experimental.pallas import tpu as pltpu
from jax.experimental.pallas import tpu_sc as plsc

_NC = 2
_NS = 16
_NW = _NC * _NS
_L = 16

_B, _T, _D = 128, 1024, 8
_SC_ROWS = 32
_TC_ROWS = _B - _SC_ROWS
_RPW = _SC_ROWS // _NW
_NT = _T // _L
_TC_RB = 16

_MAGIC = 8388608.0


def _decode(x, sel_dtype):
    one, half, zero = 1.0, 0.5, 0.0
    D = len(x)

    s = [jnp.where(x[j] >= zero, _MAGIC, -_MAGIC) for j in range(D)]
    f0 = [(x[j] + s[j]) - s[j] for j in range(D)]
    d0 = [x[j] - f0[j] for j in range(D)]
    neg = [d0[j] < zero for j in range(D)]
    f1 = [f0[j] - jnp.where(neg[j], one, zero) for j in range(D)]
    d1 = [d0[j] + jnp.where(neg[j], half, -half) for j in range(D)]

    def coset(f, d):
        p = f[0] + f[1]
        for j in range(2, D):
            p = p + f[j]
        odd = (p.astype(jnp.int32) & 1) == 1
        ad = [jnp.abs(d[j]) for j in range(D)]
        best = ad[0]
        bidx = jnp.zeros(best.shape, sel_dtype)
        for j in range(1, D):
            gt = ad[j] > best
            best = jnp.maximum(best, ad[j])
            bidx = jnp.where(gt, jnp.asarray(j, sel_dtype), bidx)
        g_out = []
        for j in range(D):
            fix = (bidx == j) & odd
            sgn = jnp.where(d[j] >= zero, one, -one)
            g_out.append(f[j] + jnp.where(fix, sgn, zero))
        dist = d[0] * d[0]
        for j in range(1, D):
            dist = dist + d[j] * d[j]
        dist = dist + jnp.where(odd, one - (best + best), zero)
        return g_out, dist

    g0, dist0 = coset(f0, d0)
    g1, dist1 = coset(f1, d1)
    pick0 = dist0 <= dist1
    return [jnp.where(pick0, g0[j], g1[j] + half) for j in range(D)]


def _e8_sc_body(x_hbm, out_hbm, xv, ov):
    wid = lax.axis_index("s") * _NC + lax.axis_index("c")
    row0 = wid * _RPW
    pltpu.sync_copy(x_hbm.at[pl.ds(row0, _RPW)], xv)

    for b in range(_RPW):
        @plsc.parallel_loop(0, _NT, 1, unroll=2)
        def step(t, b=b):
            t0 = t * _L
            x = [xv[b, j, pl.ds(t0, _L)] for j in range(_D)]
            out = _decode(x, jnp.int32)
            for j in range(_D):
                ov[b, j, pl.ds(t0, _L)] = out[j]

    pltpu.sync_copy(ov, out_hbm.at[pl.ds(row0, _RPW)])


_e8_sc = functools.partial(
    pl.kernel,
    out_type=jax.ShapeDtypeStruct((_SC_ROWS, _D, _T), jnp.float32),
    mesh=plsc.VectorSubcoreMesh(core_axis_name="c", subcore_axis_name="s"),
    scratch_types=[
        pltpu.VMEM((_RPW, _D, _T), jnp.float32),
        pltpu.VMEM((_RPW, _D, _T), jnp.float32),
    ],
    compiler_params=pltpu.CompilerParams(needs_layout_passes=False),
)(_e8_sc_body)


def _e8_tc_body(x_ref, o_ref):
    one, half, zero = 1.0, 0.5, 0.0
    x = x_ref[...]
    s = jnp.where(x >= zero, _MAGIC, -_MAGIC)
    f0 = (x + s) - s
    d0 = x - f0
    neg = d0 < zero
    f1 = f0 - jnp.where(neg, one, zero)
    d1 = d0 + jnp.where(neg, half, -half)
    jota = lax.broadcasted_iota(jnp.int32, x.shape, 1)

    def coset(f, d):
        p = jnp.sum(f, axis=1, keepdims=True)
        odd = (p.astype(jnp.int32) & 1) == 1
        ad = jnp.abs(d)
        m = jnp.max(ad, axis=1, keepdims=True)
        cand = jnp.where(ad == m, jota, _D)
        idx = jnp.min(cand, axis=1, keepdims=True)
        fix = (jota == idx) & odd
        sgn = jnp.where(d >= zero, one, -one)
        g = f + jnp.where(fix, sgn, zero)
        dist = jnp.sum(d * d, axis=1, keepdims=True)
        dist = dist + jnp.where(odd, one - (m + m), zero)
        return g, dist

    g0, dist0 = coset(f0, d0)
    g1, dist1 = coset(f1, d1)
    o_ref[...] = jnp.where(dist0 <= dist1, g0, g1 + half)


_TC_OFF = _SC_ROWS // _TC_RB

_e8_tc = pl.pallas_call(
    _e8_tc_body,
    out_shape=jax.ShapeDtypeStruct((_TC_ROWS, _D, _T), jnp.float32),
    grid=(_TC_ROWS // _TC_RB,),
    in_specs=[pl.BlockSpec((_TC_RB, _D, _T), lambda i: (i + _TC_OFF, 0, 0))],
    out_specs=pl.BlockSpec((_TC_RB, _D, _T), lambda i: (i, 0, 0)),
    compiler_params=pltpu.CompilerParams(
        dimension_semantics=("arbitrary",),
    ),
)


@jax.jit
def kernel(x):
    if x.shape[-1] != 8:
        raise ValueError(f"E8 expects [..., 8] input, got shape {x.shape}")
    xt = jnp.transpose(x, (0, 2, 1))
    y_sc = _e8_sc(xt)
    y_tc = _e8_tc(xt)
    y_t = jnp.concatenate([y_sc, y_tc], axis=0)
    return jnp.transpose(y_t, (0, 2, 1))


# speedup vs baseline: 1.8763x; 1.0447x over previous
import functools

import jax
import jax.numpy as jnp
from jax import lax
from jax.experimental import pallas as pl
from jax.experimental.pallas import tpu as pltpu
from jax.experimental.pallas import tpu_sc as plsc

_NC = 2
_NS = 16
_NW = _NC * _NS
_L = 16

_B, _T, _D = 128, 1024, 8
_SC_ROWS = 32
_TC_ROWS = _B - _SC_ROWS
_RPW = _SC_ROWS // _NW
_NT = _T // _L
_TC_RB = 16

_MAGIC = 8388608.0


def _decode(x, sel_dtype):
    one, half, zero = 1.0, 0.5, 0.0
    D = len(x)

    s = [jnp.where(x[j] >= zero, _MAGIC, -_MAGIC) for j in range(D)]
    f0 = [(x[j] + s[j]) - s[j] for j in range(D)]
    d0 = [x[j] - f0[j] for j in range(D)]
    neg = [d0[j] < zero for j in range(D)]
    f1 = [f0[j] - jnp.where(neg[j], one, zero) for j in range(D)]
    d1 = [d0[j] + jnp.where(neg[j], half, -half) for j in range(D)]

    def coset(f, d):
        p = f[0] + f[1]
        for j in range(2, D):
            p = p + f[j]
        odd = (p.astype(jnp.int32) & 1) == 1
        ad = [jnp.abs(d[j]) for j in range(D)]
        best = ad[0]
        bidx = jnp.zeros(best.shape, sel_dtype)
        for j in range(1, D):
            gt = ad[j] > best
            best = jnp.maximum(best, ad[j])
            bidx = jnp.where(gt, jnp.asarray(j, sel_dtype), bidx)
        g_out = []
        for j in range(D):
            fix = (bidx == j) & odd
            sgn = jnp.where(d[j] >= zero, one, -one)
            g_out.append(f[j] + jnp.where(fix, sgn, zero))
        dist = d[0] * d[0]
        for j in range(1, D):
            dist = dist + d[j] * d[j]
        dist = dist + jnp.where(odd, one - (best + best), zero)
        return g_out, dist

    g0, dist0 = coset(f0, d0)
    g1, dist1 = coset(f1, d1)
    pick0 = dist0 <= dist1
    return [jnp.where(pick0, g0[j], g1[j] + half) for j in range(D)]


def _e8_sc_body(x_hbm, out_hbm, xv, ov):
    wid = lax.axis_index("s") * _NC + lax.axis_index("c")
    row0 = wid * _RPW
    pltpu.sync_copy(x_hbm.at[pl.ds(row0, _RPW)], xv)

    for b in range(_RPW):
        @plsc.parallel_loop(0, _NT, 1, unroll=2)
        def step(t, b=b):
            t0 = t * _L
            x = [xv[b, j, pl.ds(t0, _L)] for j in range(_D)]
            out = _decode(x, jnp.int32)
            for j in range(_D):
                ov[b, j, pl.ds(t0, _L)] = out[j]

    pltpu.sync_copy(ov, out_hbm.at[pl.ds(row0, _RPW)])


_e8_sc = functools.partial(
    pl.kernel,
    out_type=jax.ShapeDtypeStruct((_SC_ROWS, _D, _T), jnp.float32),
    mesh=plsc.VectorSubcoreMesh(core_axis_name="c", subcore_axis_name="s"),
    scratch_types=[
        pltpu.VMEM((_RPW, _D, _T), jnp.float32),
        pltpu.VMEM((_RPW, _D, _T), jnp.float32),
    ],
    compiler_params=pltpu.CompilerParams(needs_layout_passes=False),
)(_e8_sc_body)


def _e8_tc_body(x_ref, o_ref):
    one, half, zero = 1.0, 0.5, 0.0
    x = x_ref[...]
    s = jnp.where(x >= zero, _MAGIC, -_MAGIC)
    f0 = (x + s) - s
    d0 = x - f0
    neg = d0 < zero
    f1 = f0 - jnp.where(neg, one, zero)
    d1 = d0 + jnp.where(neg, half, -half)
    rj = 7 - lax.broadcasted_iota(jnp.int32, x.shape, 1)

    def coset(f, d):
        p = jnp.sum(f, axis=1, keepdims=True)
        odd = (p.astype(jnp.int32) & 1) == 1
        ad = jnp.abs(d)
        key = (lax.bitcast_convert_type(ad, jnp.int32) & -8) | rj
        km = jnp.max(key, axis=1, keepdims=True)
        fix = (key == km) & odd
        sgn = jnp.where(d >= zero, one, -one)
        g = f + jnp.where(fix, sgn, zero)
        m = lax.bitcast_convert_type(km & -8, jnp.float32)
        c = jnp.where(odd, one - (m + m), zero)
        return g, c, ad

    g0, c0, ad0 = coset(f0, d0)
    g1, c1, _ = coset(f1, d1)
    a = jnp.sum(ad0, axis=1, keepdims=True)
    pick0 = (c0 + a) - c1 <= 2.0
    o_ref[...] = jnp.where(pick0, g0, g1 + half)


_TC_OFF = _SC_ROWS // _TC_RB

_e8_tc = pl.pallas_call(
    _e8_tc_body,
    out_shape=jax.ShapeDtypeStruct((_TC_ROWS, _D, _T), jnp.float32),
    grid=(_TC_ROWS // _TC_RB,),
    in_specs=[pl.BlockSpec((_TC_RB, _D, _T), lambda i: (i + _TC_OFF, 0, 0))],
    out_specs=pl.BlockSpec((_TC_RB, _D, _T), lambda i: (i, 0, 0)),
    compiler_params=pltpu.CompilerParams(
        dimension_semantics=("arbitrary",),
    ),
)


@jax.jit
def kernel(x):
    if x.shape[-1] != 8:
        raise ValueError(f"E8 expects [..., 8] input, got shape {x.shape}")
    xt = jnp.transpose(x, (0, 2, 1))
    y_sc = _e8_sc(xt)
    y_tc = _e8_tc(xt)
    y_t = jnp.concatenate([y_sc, y_tc], axis=0)
    return jnp.transpose(y_t, (0, 2, 1))


# speedup vs baseline: 1.8832x; 1.0037x over previous
import functools

import jax
import jax.numpy as jnp
from jax import lax
from jax.experimental import pallas as pl
from jax.experimental.pallas import tpu as pltpu
from jax.experimental.pallas import tpu_sc as plsc

_NC = 2
_NS = 16
_NW = _NC * _NS
_L = 16

_B, _T, _D = 128, 1024, 8
_SC_ROWS = 32
_TC_ROWS = _B - _SC_ROWS
_RPW = _SC_ROWS // _NW
_NT = _T // _L
_TC_RB = 16

_MAGIC = 8388608.0


def _decode(x, sel_dtype):
    one, half, zero = 1.0, 0.5, 0.0
    D = len(x)

    s = [jnp.where(x[j] >= zero, _MAGIC, -_MAGIC) for j in range(D)]
    f0 = [(x[j] + s[j]) - s[j] for j in range(D)]
    d0 = [x[j] - f0[j] for j in range(D)]
    neg = [d0[j] < zero for j in range(D)]
    f1 = [f0[j] - jnp.where(neg[j], one, zero) for j in range(D)]
    d1 = [d0[j] + jnp.where(neg[j], half, -half) for j in range(D)]

    def coset(f, d):
        p = f[0] + f[1]
        for j in range(2, D):
            p = p + f[j]
        odd = (p.astype(jnp.int32) & 1) == 1
        ad = [jnp.abs(d[j]) for j in range(D)]
        best = ad[0]
        bidx = jnp.zeros(best.shape, sel_dtype)
        for j in range(1, D):
            gt = ad[j] > best
            best = jnp.maximum(best, ad[j])
            bidx = jnp.where(gt, jnp.asarray(j, sel_dtype), bidx)
        g_out = []
        for j in range(D):
            fix = (bidx == j) & odd
            sgn = jnp.where(d[j] >= zero, one, -one)
            g_out.append(f[j] + jnp.where(fix, sgn, zero))
        dist = d[0] * d[0]
        for j in range(1, D):
            dist = dist + d[j] * d[j]
        dist = dist + jnp.where(odd, one - (best + best), zero)
        return g_out, dist

    g0, dist0 = coset(f0, d0)
    g1, dist1 = coset(f1, d1)
    pick0 = dist0 <= dist1
    return [jnp.where(pick0, g0[j], g1[j] + half) for j in range(D)]


def _e8_sc_body(x_hbm, out_hbm, xv, ov):
    wid = lax.axis_index("s") * _NC + lax.axis_index("c")
    row0 = wid * _RPW
    pltpu.sync_copy(x_hbm.at[pl.ds(row0, _RPW)], xv)

    for b in range(_RPW):
        @plsc.parallel_loop(0, _NT, 1, unroll=2)
        def step(t, b=b):
            t0 = t * _L
            x = [xv[b, j, pl.ds(t0, _L)] for j in range(_D)]
            out = _decode(x, jnp.int32)
            for j in range(_D):
                ov[b, j, pl.ds(t0, _L)] = out[j]

    pltpu.sync_copy(ov, out_hbm.at[pl.ds(row0, _RPW)])


_e8_sc = functools.partial(
    pl.kernel,
    out_type=jax.ShapeDtypeStruct((_SC_ROWS, _D, _T), jnp.float32),
    mesh=plsc.VectorSubcoreMesh(core_axis_name="c", subcore_axis_name="s"),
    scratch_types=[
        pltpu.VMEM((_RPW, _D, _T), jnp.float32),
        pltpu.VMEM((_RPW, _D, _T), jnp.float32),
    ],
    compiler_params=pltpu.CompilerParams(needs_layout_passes=False),
)(_e8_sc_body)


def _e8_tc_body(x_ref, o_ref):
    one, half, zero = 1.0, 0.5, 0.0
    x = x_ref[...]
    s = jnp.where(x >= zero, _MAGIC, -_MAGIC)
    f0 = (x + s) - s
    d0 = x - f0
    neg = d0 < zero
    negf = jnp.where(neg, one, zero)
    t = half - negf
    ad0 = jnp.abs(d0)
    ad1 = half - ad0
    u = negf * 16.0 + ad0
    rj = 7 - lax.broadcasted_iota(jnp.int32, x.shape, 1)
    key0 = (lax.bitcast_convert_type(ad0, jnp.int32) & -8) | rj
    key1 = (lax.bitcast_convert_type(ad1, jnp.int32) & -8) | rj
    p0 = jnp.sum(f0, axis=1, keepdims=True)
    S = jnp.sum(u, axis=1, keepdims=True)
    km0 = jnp.max(key0, axis=1, keepdims=True)
    km1 = jnp.max(key1, axis=1, keepdims=True)
    cntf = (S * 0.0625).astype(jnp.int32).astype(jnp.float32)
    A = S - cntf * 16.0
    odd0 = (p0.astype(jnp.int32) & 1) == 1
    odd1 = ((p0 - cntf).astype(jnp.int32) & 1) == 1
    m0 = lax.bitcast_convert_type(km0 & -8, jnp.float32)
    m1 = lax.bitcast_convert_type(km1 & -8, jnp.float32)
    c0 = jnp.where(odd0, one - (m0 + m0), zero)
    c1 = jnp.where(odd1, one - (m1 + m1), zero)
    pick0 = (c0 + A) - c1 <= 2.0
    sgn0 = jnp.where(neg, -one, one)
    fix0 = (key0 == km0) & odd0
    fix1 = (key1 == km1) & odd1
    g0 = f0 + jnp.where(fix0, sgn0, zero)
    out1 = (f0 + t) - jnp.where(fix1, sgn0, zero)
    o_ref[...] = jnp.where(pick0, g0, out1)


_TC_OFF = _SC_ROWS // _TC_RB

_e8_tc = pl.pallas_call(
    _e8_tc_body,
    out_shape=jax.ShapeDtypeStruct((_TC_ROWS, _D, _T), jnp.float32),
    grid=(_TC_ROWS // _TC_RB,),
    in_specs=[pl.BlockSpec((_TC_RB, _D, _T), lambda i: (i + _TC_OFF, 0, 0))],
    out_specs=pl.BlockSpec((_TC_RB, _D, _T), lambda i: (i, 0, 0)),
    compiler_params=pltpu.CompilerParams(
        dimension_semantics=("arbitrary",),
    ),
)


@jax.jit
def kernel(x):
    if x.shape[-1] != 8:
        raise ValueError(f"E8 expects [..., 8] input, got shape {x.shape}")
    xt = jnp.transpose(x, (0, 2, 1))
    y_sc = _e8_sc(xt)
    y_tc = _e8_tc(xt)
    y_t = jnp.concatenate([y_sc, y_tc], axis=0)
    return jnp.transpose(y_t, (0, 2, 1))


# speedup vs baseline: 1.8870x; 1.0020x over previous
import functools

import jax
import jax.numpy as jnp
from jax import lax
from jax.experimental import pallas as pl
from jax.experimental.pallas import tpu as pltpu
from jax.experimental.pallas import tpu_sc as plsc

_NC = 2
_NS = 16
_NW = _NC * _NS
_L = 16

_B, _T, _D = 128, 1024, 8
_SC_ROWS = 32
_TC_ROWS = _B - _SC_ROWS
_RPW = _SC_ROWS // _NW
_NT = _T // _L
_TC_RB = 16

_MAGIC = 8388608.0


def _decode(x):
    one, half, zero = 1.0, 0.5, 0.0
    D = len(x)

    s = [jnp.where(x[j] >= zero, _MAGIC, -_MAGIC) for j in range(D)]
    f0 = [(x[j] + s[j]) - s[j] for j in range(D)]
    d0 = [x[j] - f0[j] for j in range(D)]
    neg = [d0[j] < zero for j in range(D)]
    negf = [jnp.where(neg[j], one, zero) for j in range(D)]
    t = [half - negf[j] for j in range(D)]
    ad0 = [jnp.abs(d0[j]) for j in range(D)]
    ad1 = [half - ad0[j] for j in range(D)]
    u = [negf[j] * 16.0 + ad0[j] for j in range(D)]
    key0 = [(lax.bitcast_convert_type(ad0[j], jnp.int32) & -8) | (7 - j)
            for j in range(D)]
    key1 = [(lax.bitcast_convert_type(ad1[j], jnp.int32) & -8) | (7 - j)
            for j in range(D)]

    def tree_sum(v):
        a = [v[0] + v[1], v[2] + v[3], v[4] + v[5], v[6] + v[7]]
        b = [a[0] + a[1], a[2] + a[3]]
        return b[0] + b[1]

    def tree_max(v):
        a = [jnp.maximum(v[0], v[1]), jnp.maximum(v[2], v[3]),
             jnp.maximum(v[4], v[5]), jnp.maximum(v[6], v[7])]
        b = [jnp.maximum(a[0], a[1]), jnp.maximum(a[2], a[3])]
        return jnp.maximum(b[0], b[1])

    p0 = tree_sum(f0)
    S = tree_sum(u)
    km0 = tree_max(key0)
    km1 = tree_max(key1)
    cntf = (S * 0.0625).astype(jnp.int32).astype(jnp.float32)
    A = S - cntf * 16.0
    odd0 = (p0.astype(jnp.int32) & 1) == 1
    odd1 = ((p0 - cntf).astype(jnp.int32) & 1) == 1
    m0 = lax.bitcast_convert_type(km0 & -8, jnp.float32)
    m1 = lax.bitcast_convert_type(km1 & -8, jnp.float32)
    c0 = jnp.where(odd0, one - (m0 + m0), zero)
    c1 = jnp.where(odd1, one - (m1 + m1), zero)
    pick0 = (c0 + A) - c1 <= 2.0
    out = []
    for j in range(D):
        sgn = jnp.where(neg[j], -one, one)
        fix0 = (key0[j] == km0) & odd0
        fix1 = (key1[j] == km1) & odd1
        g0 = f0[j] + jnp.where(fix0, sgn, zero)
        out1 = (f0[j] + t[j]) - jnp.where(fix1, sgn, zero)
        out.append(jnp.where(pick0, g0, out1))
    return out


def _e8_sc_body(x_hbm, out_hbm, xv, ov):
    wid = lax.axis_index("s") * _NC + lax.axis_index("c")
    row0 = wid * _RPW
    pltpu.sync_copy(x_hbm.at[pl.ds(row0, _RPW)], xv)

    for b in range(_RPW):
        @plsc.parallel_loop(0, _NT, 1, unroll=2)
        def step(t, b=b):
            t0 = t * _L
            x = [xv[b, j, pl.ds(t0, _L)] for j in range(_D)]
            out = _decode(x)
            for j in range(_D):
                ov[b, j, pl.ds(t0, _L)] = out[j]

    pltpu.sync_copy(ov, out_hbm.at[pl.ds(row0, _RPW)])


_e8_sc = functools.partial(
    pl.kernel,
    out_type=jax.ShapeDtypeStruct((_SC_ROWS, _D, _T), jnp.float32),
    mesh=plsc.VectorSubcoreMesh(core_axis_name="c", subcore_axis_name="s"),
    scratch_types=[
        pltpu.VMEM((_RPW, _D, _T), jnp.float32),
        pltpu.VMEM((_RPW, _D, _T), jnp.float32),
    ],
    compiler_params=pltpu.CompilerParams(needs_layout_passes=False),
)(_e8_sc_body)


def _e8_tc_body(x_ref, o_ref):
    one, half, zero = 1.0, 0.5, 0.0
    x = x_ref[...]
    f0 = jnp.round(x)
    d0 = x - f0
    neg = d0 < zero
    negf = jnp.where(neg, one, zero)
    t = half - negf
    ad0 = jnp.abs(d0)
    ad1 = half - ad0
    u = negf * 16.0 + ad0
    rj = 7 - lax.broadcasted_iota(jnp.int32, x.shape, 1)
    key0 = (lax.bitcast_convert_type(ad0, jnp.int32) & -8) | rj
    key1 = (lax.bitcast_convert_type(ad1, jnp.int32) & -8) | rj
    p0 = jnp.sum(f0, axis=1, keepdims=True)
    S = jnp.sum(u, axis=1, keepdims=True)
    km0 = jnp.max(key0, axis=1, keepdims=True)
    km1 = jnp.max(key1, axis=1, keepdims=True)
    cntf = (S * 0.0625).astype(jnp.int32).astype(jnp.float32)
    A = S - cntf * 16.0
    odd0 = (p0.astype(jnp.int32) & 1) == 1
    odd1 = ((p0 - cntf).astype(jnp.int32) & 1) == 1
    m0 = lax.bitcast_convert_type(km0 & -8, jnp.float32)
    m1 = lax.bitcast_convert_type(km1 & -8, jnp.float32)
    c0 = jnp.where(odd0, one - (m0 + m0), zero)
    c1 = jnp.where(odd1, one - (m1 + m1), zero)
    pick0 = (c0 + A) - c1 <= 2.0
    sgn0 = jnp.where(neg, -one, one)
    fix0 = (key0 == km0) & odd0
    fix1 = (key1 == km1) & odd1
    g0 = f0 + jnp.where(fix0, sgn0, zero)
    out1 = (f0 + t) - jnp.where(fix1, sgn0, zero)
    o_ref[...] = jnp.where(pick0, g0, out1)


_TC_OFF = _SC_ROWS // _TC_RB

_e8_tc = pl.pallas_call(
    _e8_tc_body,
    out_shape=jax.ShapeDtypeStruct((_TC_ROWS, _D, _T), jnp.float32),
    grid=(_TC_ROWS // _TC_RB,),
    in_specs=[pl.BlockSpec((_TC_RB, _D, _T), lambda i: (i + _TC_OFF, 0, 0))],
    out_specs=pl.BlockSpec((_TC_RB, _D, _T), lambda i: (i, 0, 0)),
    compiler_params=pltpu.CompilerParams(
        dimension_semantics=("arbitrary",),
    ),
)


@jax.jit
def kernel(x):
    if x.shape[-1] != 8:
        raise ValueError(f"E8 expects [..., 8] input, got shape {x.shape}")
    xt = jnp.transpose(x, (0, 2, 1))
    y_sc = _e8_sc(xt)
    y_tc = _e8_tc(xt)
    y_t = jnp.concatenate([y_sc, y_tc], axis=0)
    return jnp.transpose(y_t, (0, 2, 1))


# speedup vs baseline: 2.0546x; 1.0888x over previous
import functools

import jax
import jax.numpy as jnp
from jax import lax
from jax.experimental import pallas as pl
from jax.experimental.pallas import tpu as pltpu
from jax.experimental.pallas import tpu_sc as plsc

_NC = 2
_NS = 16
_NW = _NC * _NS
_L = 16

_B, _T, _D = 128, 1024, 8
_SC_ROWS = 32
_TC_ROWS = _B - _SC_ROWS
_RPW = _SC_ROWS // _NW
_NT = _T // _L
_TC_RB = 16

_MAGIC = 8388608.0


def _decode(x):
    one, half, zero = 1.0, 0.5, 0.0
    D = len(x)

    s = [jnp.where(x[j] >= zero, _MAGIC, -_MAGIC) for j in range(D)]
    f0 = [(x[j] + s[j]) - s[j] for j in range(D)]
    d0 = [x[j] - f0[j] for j in range(D)]
    neg = [d0[j] < zero for j in range(D)]
    negf = [jnp.where(neg[j], one, zero) for j in range(D)]
    t = [half - negf[j] for j in range(D)]
    ad0 = [jnp.abs(d0[j]) for j in range(D)]
    ad1 = [half - ad0[j] for j in range(D)]
    u = [negf[j] * 16.0 + ad0[j] for j in range(D)]
    key0 = [(lax.bitcast_convert_type(ad0[j], jnp.int32) & -8) | (7 - j)
            for j in range(D)]
    key1 = [(lax.bitcast_convert_type(ad1[j], jnp.int32) & -8) | (7 - j)
            for j in range(D)]

    def tree_sum(v):
        a = [v[0] + v[1], v[2] + v[3], v[4] + v[5], v[6] + v[7]]
        b = [a[0] + a[1], a[2] + a[3]]
        return b[0] + b[1]

    def tree_max(v):
        a = [jnp.maximum(v[0], v[1]), jnp.maximum(v[2], v[3]),
             jnp.maximum(v[4], v[5]), jnp.maximum(v[6], v[7])]
        b = [jnp.maximum(a[0], a[1]), jnp.maximum(a[2], a[3])]
        return jnp.maximum(b[0], b[1])

    p0 = tree_sum(f0)
    S = tree_sum(u)
    km0 = tree_max(key0)
    km1 = tree_max(key1)
    cntf = (S * 0.0625).astype(jnp.int32).astype(jnp.float32)
    A = S - cntf * 16.0
    odd0 = (p0.astype(jnp.int32) & 1) == 1
    odd1 = ((p0 - cntf).astype(jnp.int32) & 1) == 1
    m0 = lax.bitcast_convert_type(km0 & -8, jnp.float32)
    m1 = lax.bitcast_convert_type(km1 & -8, jnp.float32)
    c0 = jnp.where(odd0, one - (m0 + m0), zero)
    c1 = jnp.where(odd1, one - (m1 + m1), zero)
    pick0 = (c0 + A) - c1 <= 2.0
    out = []
    for j in range(D):
        sgn = jnp.where(neg[j], -one, one)
        fix0 = (key0[j] == km0) & odd0
        fix1 = (key1[j] == km1) & odd1
        g0 = f0[j] + jnp.where(fix0, sgn, zero)
        out1 = (f0[j] + t[j]) - jnp.where(fix1, sgn, zero)
        out.append(jnp.where(pick0, g0, out1))
    return out


def _e8_sc_body(x_hbm, out_hbm, xv, ov):
    wid = lax.axis_index("s") * _NC + lax.axis_index("c")
    row0 = wid * _RPW
    pltpu.sync_copy(x_hbm.at[pl.ds(row0, _RPW)], xv)

    for b in range(_RPW):
        @plsc.parallel_loop(0, _NT, 1, unroll=2)
        def step(t, b=b):
            t0 = t * _L
            x = [xv[b, j, pl.ds(t0, _L)] for j in range(_D)]
            out = _decode(x)
            for j in range(_D):
                ov[b, j, pl.ds(t0, _L)] = out[j]

    pltpu.sync_copy(ov, out_hbm.at[pl.ds(row0, _RPW)])


_e8_sc = functools.partial(
    pl.kernel,
    out_type=jax.ShapeDtypeStruct((_SC_ROWS, _D, _T), jnp.float32),
    mesh=plsc.VectorSubcoreMesh(core_axis_name="c", subcore_axis_name="s"),
    scratch_types=[
        pltpu.VMEM((_RPW, _D, _T), jnp.float32),
        pltpu.VMEM((_RPW, _D, _T), jnp.float32),
    ],
    compiler_params=pltpu.CompilerParams(needs_layout_passes=False),
)(_e8_sc_body)


def _e8_tc_body(x_ref, o_ref):
    one, half, zero = 1.0, 0.5, 0.0
    x = x_ref[...]
    f0 = jnp.round(x)
    d0 = x - f0
    neg = d0 < zero
    negf = jnp.where(neg, one, zero)
    t = half - negf
    ad0 = jnp.abs(d0)
    ad1 = half - ad0
    u = negf * 16.0 + ad0
    rj = 7 - lax.broadcasted_iota(jnp.int32, x.shape, 1)
    key0 = (lax.bitcast_convert_type(ad0, jnp.int32) & -8) | rj
    key1 = (lax.bitcast_convert_type(ad1, jnp.int32) & -8) | rj
    p0 = jnp.sum(f0, axis=1, keepdims=True)
    S = jnp.sum(u, axis=1, keepdims=True)
    km0 = jnp.max(key0, axis=1, keepdims=True)
    km1 = jnp.max(key1, axis=1, keepdims=True)
    cntf = (S * 0.0625).astype(jnp.int32).astype(jnp.float32)
    A = S - cntf * 16.0
    odd0 = (p0.astype(jnp.int32) & 1) == 1
    odd1 = ((p0 - cntf).astype(jnp.int32) & 1) == 1
    m0 = lax.bitcast_convert_type(km0 & -8, jnp.float32)
    m1 = lax.bitcast_convert_type(km1 & -8, jnp.float32)
    c0 = jnp.where(odd0, one - (m0 + m0), zero)
    c1 = jnp.where(odd1, one - (m1 + m1), zero)
    pick0 = (c0 + A) - c1 <= 2.0
    sgn0 = jnp.where(neg, -one, one)
    fix0 = (key0 == km0) & odd0
    fix1 = (key1 == km1) & odd1
    g0 = f0 + jnp.where(fix0, sgn0, zero)
    out1 = (f0 + t) - jnp.where(fix1, sgn0, zero)
    o_ref[...] = jnp.where(pick0, g0, out1)


_TC_OFF = _SC_ROWS // _TC_RB

_e8_tc = pl.pallas_call(
    _e8_tc_body,
    out_shape=jax.ShapeDtypeStruct((_B, _D, _T), jnp.float32),
    grid=(_TC_ROWS // _TC_RB,),
    in_specs=[pl.BlockSpec((_TC_RB, _D, _T), lambda i: (i + _TC_OFF, 0, 0))],
    out_specs=pl.BlockSpec((_TC_RB, _D, _T), lambda i: (i + _TC_OFF, 0, 0)),
    compiler_params=pltpu.CompilerParams(
        dimension_semantics=("arbitrary",),
    ),
)


@jax.jit
def kernel(x):
    if x.shape[-1] != 8:
        raise ValueError(f"E8 expects [..., 8] input, got shape {x.shape}")
    xt = jnp.transpose(x, (0, 2, 1))
    y_sc = _e8_sc(xt)
    y_tc = _e8_tc(xt)
    y_t = lax.dynamic_update_slice(y_tc, y_sc, (0, 0, 0))
    return jnp.transpose(y_t, (0, 2, 1))
